# baseline (speedup 1.0000x reference)
import functools

import jax
import jax.numpy as jnp
from jax import lax
from jax.experimental import pallas as pl
from jax.experimental.pallas import tpu as pltpu
from jax.experimental.pallas import tpu_sc as plsc

B = 16384
D = 32
NROWS = 1000000

_info = plsc.get_sparse_core_info()
NC = _info.num_cores
NS = _info.num_subcores
L = _info.num_lanes
NW = NC * NS
BPW = B // NW
NG = BPW // L
DBLK = 4
NBLK = D // DBLK

_mesh = plsc.VectorSubcoreMesh(core_axis_name="c", subcore_axis_name="s")


@functools.partial(
    pl.kernel,
    mesh=_mesh,
    compiler_params=pltpu.CompilerParams(
        needs_layout_passes=False, use_tc_tiling_on_sc=False),
    out_type=jax.ShapeDtypeStruct((B,), jnp.float32),
    scratch_types=[
        pltpu.VMEM((BPW,), jnp.int32),
        pltpu.VMEM((BPW,), jnp.int32),
        pltpu.VMEM((NBLK, DBLK * BPW), jnp.int32),
        pltpu.VMEM((NBLK, DBLK * BPW), jnp.int32),
        pltpu.VMEM((NBLK, DBLK * BPW), jnp.float32),
        pltpu.VMEM((NBLK, DBLK * BPW), jnp.float32),
        pltpu.VMEM((BPW,), jnp.float32),
        pltpu.VMEM((BPW,), jnp.float32),
        pltpu.VMEM((BPW,), jnp.float32),
        pltpu.VMEM((BPW,), jnp.float32),
        pltpu.SemaphoreType.DMA,
        pltpu.SemaphoreType.DMA,
    ],
)
def _mf_kernel(uid_hbm, iid_hbm, avg_hbm, p_hbm, q_hbm, ub_hbm, ib_hbm,
               out_hbm, uidx, iidx, pidx, qidx, pdv, qdv, ubv, ibv, avgv,
               outv, gsem, bsem):
    wid = lax.axis_index("s") * NC + lax.axis_index("c")
    base = wid * BPW

    pltpu.sync_copy(uid_hbm.at[pl.ds(base, BPW)], uidx)
    pltpu.sync_copy(iid_hbm.at[pl.ds(base, BPW)], iidx)
    cu = pltpu.async_copy(ub_hbm.at[uidx], ubv, bsem)
    ci = pltpu.async_copy(ib_hbm.at[iidx], ibv, bsem)

    copies = []
    for blk in range(NBLK):
        for dd in range(DBLK):
            d = blk * DBLK + dd
            def fill(g, _, dd=dd, d=d):
                o = g * L
                u = uidx[pl.ds(o, L)]
                i = iidx[pl.ds(o, L)]
                pidx[blk, pl.ds(dd * BPW + o, L)] = u + d * NROWS
                qidx[blk, pl.ds(dd * BPW + o, L)] = i + d * NROWS
                return 0
            lax.fori_loop(0, NG, fill, 0)
        copies.append(
            pltpu.async_copy(p_hbm.at[pidx.at[blk]], pdv.at[blk], gsem))
        copies.append(
            pltpu.async_copy(q_hbm.at[qidx.at[blk]], qdv.at[blk], gsem))

    pltpu.sync_copy(avg_hbm.at[0, pl.ds(base, BPW)], avgv)
    cu.wait()
    ci.wait()
    for c in copies:
        c.wait()

    def body(g, _):
        o = g * L
        acc = avgv[pl.ds(o, L)] + ubv[pl.ds(o, L)] + ibv[pl.ds(o, L)]
        for d in range(D):
            blk, dd = divmod(d, DBLK)
            acc += (pdv[blk, pl.ds(dd * BPW + o, L)]
                    * qdv[blk, pl.ds(dd * BPW + o, L)])
        outv[pl.ds(o, L)] = acc
        return 0

    lax.fori_loop(0, NG, body, 0)
    pltpu.sync_copy(outv, out_hbm.at[pl.ds(base, BPW)])


def kernel(user_id, item_id, avg_score, p, q, user_bias, item_bias):
    out = _mf_kernel(user_id, item_id, avg_score.T,
                     p.T.reshape(D * NROWS), q.T.reshape(D * NROWS),
                     user_bias.T.reshape(NROWS), item_bias.T.reshape(NROWS))
    return out.reshape(B, 1)
